# baseline (speedup 1.0000x reference)
import jax
import jax.numpy as jnp
from jax.experimental import pallas as pl
from jax.experimental.pallas import tpu as pltpu

B, S, D = 1, 2048, 768
H, DH = 12, 64
LAT, RD = 384, 64
E, K, DFF = 8, 2, 512
T = B * S

TM = 256
TQ = 128
NT = T // TM
NQ = T // TQ


def _dot16(a, bm):
    if a.dtype != jnp.bfloat16:
        a = a.astype(jnp.bfloat16)
    if bm.dtype != jnp.bfloat16:
        bm = bm.astype(jnp.bfloat16)
    return jnp.dot(a, bm, preferred_element_type=jnp.float32)


def _ln_apply(xv, mu, var, w, b):
    return (xv - mu) / jnp.sqrt(var + 1e-5) * w + b


def _proj_in_kernel(x_ref, mu_ref, var_ref, w_ref, b_ref, wcat_ref,
                    wkvkr_ref, wqqr_ref, okv_ref, oq_ref):
    h = _ln_apply(x_ref[...], mu_ref[...], var_ref[...], w_ref[...],
                  b_ref[...])
    c = _dot16(h, wcat_ref[...])
    okv_ref[...] = _dot16(c[:, :LAT], wkvkr_ref[...])
    oq_ref[...] = _dot16(c[:, LAT:], wqqr_ref[...])


def _flash_kernel(q_ref, k_ref, v_ref, o_ref, p_buf):
    i = pl.program_id(1)
    q = q_ref[0]
    rows = i * TQ + jax.lax.broadcasted_iota(jnp.int32, (TQ, TQ), 0)

    def pass_a(j, m):
        kb = k_ref[0, pl.ds(j * TQ, TQ), :]
        s = jax.lax.dot_general(q.astype(jnp.bfloat16),
                                kb.astype(jnp.bfloat16),
                                (((1,), (1,)), ((), ())),
                                preferred_element_type=jnp.float32)
        s = s * (DH ** -0.5)
        cols = j * TQ + jax.lax.broadcasted_iota(jnp.int32, (TQ, TQ), 1)
        s = jnp.where(cols <= rows, s, -1e30)
        p_buf[j] = s
        return jnp.maximum(m, jnp.max(s, axis=-1, keepdims=True))

    m = jax.lax.fori_loop(0, i + 1, pass_a,
                          jnp.full((TQ, 1), -1e30, jnp.float32))

    def pass_b(j, l):
        p = jnp.exp(p_buf[j] - m)
        p_buf[j] = p
        return l + jnp.sum(p, axis=-1, keepdims=True)

    l = jax.lax.fori_loop(0, i + 1, pass_b, jnp.zeros((TQ, 1), jnp.float32))

    def pass_c(j, acc):
        vb = v_ref[0, pl.ds(j * TQ, TQ), :]
        pn = p_buf[j] / l
        return acc + _dot16(pn, vb)

    acc = jax.lax.fori_loop(0, i + 1, pass_c,
                            jnp.zeros((TQ, 128), jnp.float32))
    o_ref[0] = acc[:, :DH]


def _attnout_kernel(att_ref, x_ref, wout_ref, x2_ref):
    x2_ref[...] = x_ref[...] + _dot16(att_ref[...], wout_ref[...])


def _router_kernel(x2_ref, mu_ref, var_ref, w2_ref, b2_ref, ct_ref,
                   h2_ref, comb_ref, cnt_ref):
    h2 = _ln_apply(x2_ref[...], mu_ref[...], var_ref[...], w2_ref[...],
                   b2_ref[...])
    h2_ref[...] = h2
    logits = _dot16(h2, ct_ref[...])[:, :E]
    aff = jax.nn.sigmoid(logits)
    cols = jax.lax.broadcasted_iota(jnp.int32, (TM, E), 1)
    big = jnp.int32(E)
    v1 = jnp.max(aff, axis=-1, keepdims=True)
    i1 = jnp.min(jnp.where(aff == v1, cols, big), axis=-1, keepdims=True)
    aff2 = jnp.where(cols == i1, -1.0, aff)
    v2 = jnp.max(aff2, axis=-1, keepdims=True)
    i2 = jnp.min(jnp.where(aff2 == v2, cols, big), axis=-1, keepdims=True)
    e2 = jnp.exp(v2 - v1)
    w1 = 1.0 / (1.0 + e2)
    w2 = e2 / (1.0 + e2)
    oh1 = (cols == i1)
    oh2 = (cols == i2)
    comb_ref[...] = (oh1.astype(jnp.float32) * w1 +
                     oh2.astype(jnp.float32) * w2)
    cnt_ref[0] = jnp.sum(oh1.astype(jnp.int32) + oh2.astype(jnp.int32),
                         axis=0, keepdims=True)


def _moe_kernel(h2_ref, x2_ref, comb_ref, wg_ref, wu_ref, wd_ref, o_ref):
    h2 = h2_ref[...]
    acc = x2_ref[...]
    comb = comb_ref[...]
    for e in range(E):
        g = _dot16(h2, wg_ref[e])
        u = _dot16(h2, wu_ref[e])
        a = g * jax.nn.sigmoid(g) * u
        y = _dot16(a, wd_ref[e])
        acc = acc + comb[:, e:e + 1] * y
    o_ref[...] = acc


def _rope(xv, cosv, sinv):
    xe = xv[..., 0::2]
    xo = xv[..., 1::2]
    re = xe * cosv - xo * sinv
    ro = xe * sinv + xo * cosv
    return jnp.stack([re, ro], axis=-1).reshape(xv.shape)


def _row_stats(xv):
    mu = jnp.mean(xv, axis=-1, keepdims=True)
    var = jnp.mean((xv - mu) ** 2, axis=-1, keepdims=True)
    return mu, var


@jax.jit
def kernel(x, ln1_w, ln1_b, ln2_w, ln2_b, W_ckv, W_kv, W_cq, W_q, W_qr,
           W_kr, W_out, centroids, Wg, Wu, Wd):
    xf = x.reshape(T, D)
    bf = jnp.bfloat16
    w_cat = jnp.concatenate([W_ckv, W_cq], axis=1).astype(bf)
    w_kvkr = jnp.concatenate(
        [W_kv, W_kr, jnp.zeros((LAT, 128 - RD), jnp.float32)],
        axis=1).astype(bf)
    w_qqr = jnp.concatenate([W_q, W_qr], axis=1).astype(bf)
    mu1, var1 = _row_stats(xf)

    okv, oq = pl.pallas_call(
        _proj_in_kernel,
        grid=(NT,),
        in_specs=[
            pl.BlockSpec((TM, D), lambda i: (i, 0)),
            pl.BlockSpec((TM, 1), lambda i: (i, 0)),
            pl.BlockSpec((TM, 1), lambda i: (i, 0)),
            pl.BlockSpec((1, D), lambda i: (0, 0)),
            pl.BlockSpec((1, D), lambda i: (0, 0)),
            pl.BlockSpec((D, 2 * LAT), lambda i: (0, 0)),
            pl.BlockSpec((LAT, 2 * D + 128), lambda i: (0, 0)),
            pl.BlockSpec((LAT, D + H * RD), lambda i: (0, 0)),
        ],
        out_specs=[
            pl.BlockSpec((TM, 2 * D + 128), lambda i: (i, 0)),
            pl.BlockSpec((TM, D + H * RD), lambda i: (i, 0)),
        ],
        out_shape=[
            jax.ShapeDtypeStruct((T, 2 * D + 128), jnp.float32),
            jax.ShapeDtypeStruct((T, D + H * RD), jnp.float32),
        ],
    )(xf, mu1, var1, ln1_w.reshape(1, D), ln1_b.reshape(1, D), w_cat,
      w_kvkr, w_qqr)

    kv = okv[:, :2 * D]
    kr = okv[:, 2 * D:2 * D + RD]
    k_c = kv[:, :D].reshape(T, H, DH)
    v = kv[:, D:].reshape(T, H, DH)
    q_c = oq[:, :D].reshape(T, H, DH)
    q_r = oq[:, D:].reshape(T, H, RD)

    inv_freq = 1.0 / (10000.0 ** (jnp.arange(0, RD, 2, dtype=jnp.float32) / RD))
    freqs = jnp.outer(jnp.arange(S, dtype=jnp.float32), inv_freq)
    cosv = jnp.cos(freqs)
    sinv = jnp.sin(freqs)

    q_rot = _rope(q_r, cosv[:, None, :], sinv[:, None, :])
    k_rot = _rope(kr, cosv, sinv)

    q = jnp.concatenate([q_c, q_rot], axis=-1).transpose(1, 0, 2)
    k = jnp.concatenate(
        [k_c, jnp.broadcast_to(k_rot[:, None, :], (T, H, RD))],
        axis=-1).transpose(1, 0, 2)
    vt = jnp.concatenate(
        [v.transpose(1, 0, 2), jnp.zeros((H, T, 128 - DH), jnp.float32)],
        axis=-1)

    att = pl.pallas_call(
        _flash_kernel,
        grid=(H, NQ),
        in_specs=[
            pl.BlockSpec((1, TQ, DH + RD), lambda h, i: (h, i, 0)),
            pl.BlockSpec((1, T, DH + RD), lambda h, i: (h, 0, 0)),
            pl.BlockSpec((1, T, 128), lambda h, i: (h, 0, 0)),
        ],
        out_specs=pl.BlockSpec((1, TQ, DH), lambda h, i: (h, i, 0)),
        out_shape=jax.ShapeDtypeStruct((H, T, DH), jnp.float32),
        scratch_shapes=[pltpu.VMEM((NQ, TQ, TQ), jnp.float32)],
    )(q, k, vt)

    att2d = att.transpose(1, 0, 2).reshape(T, D)

    x2 = pl.pallas_call(
        _attnout_kernel,
        grid=(NT,),
        in_specs=[
            pl.BlockSpec((TM, D), lambda i: (i, 0)),
            pl.BlockSpec((TM, D), lambda i: (i, 0)),
            pl.BlockSpec((D, D), lambda i: (0, 0)),
        ],
        out_specs=pl.BlockSpec((TM, D), lambda i: (i, 0)),
        out_shape=jax.ShapeDtypeStruct((T, D), jnp.float32),
    )(att2d, xf, W_out.astype(bf))

    mu2, var2 = _row_stats(x2)
    ct_pad = jnp.concatenate(
        [centroids.T, jnp.zeros((D, 128 - E), jnp.float32)],
        axis=1).astype(bf)

    h2, comb, cnt = pl.pallas_call(
        _router_kernel,
        grid=(NT,),
        in_specs=[
            pl.BlockSpec((TM, D), lambda i: (i, 0)),
            pl.BlockSpec((TM, 1), lambda i: (i, 0)),
            pl.BlockSpec((TM, 1), lambda i: (i, 0)),
            pl.BlockSpec((1, D), lambda i: (0, 0)),
            pl.BlockSpec((1, D), lambda i: (0, 0)),
            pl.BlockSpec((D, 128), lambda i: (0, 0)),
        ],
        out_specs=[
            pl.BlockSpec((TM, D), lambda i: (i, 0)),
            pl.BlockSpec((TM, E), lambda i: (i, 0)),
            pl.BlockSpec((1, 1, E), lambda i: (i, 0, 0)),
        ],
        out_shape=[
            jax.ShapeDtypeStruct((T, D), jnp.float32),
            jax.ShapeDtypeStruct((T, E), jnp.float32),
            jax.ShapeDtypeStruct((NT, 1, E), jnp.int32),
        ],
    )(x2, mu2, var2, ln2_w.reshape(1, D), ln2_b.reshape(1, D), ct_pad)

    out = pl.pallas_call(
        _moe_kernel,
        grid=(NT,),
        in_specs=[
            pl.BlockSpec((TM, D), lambda i: (i, 0)),
            pl.BlockSpec((TM, D), lambda i: (i, 0)),
            pl.BlockSpec((TM, E), lambda i: (i, 0)),
            pl.BlockSpec((E, D, DFF), lambda i: (0, 0, 0)),
            pl.BlockSpec((E, D, DFF), lambda i: (0, 0, 0)),
            pl.BlockSpec((E, DFF, D), lambda i: (0, 0, 0)),
        ],
        out_specs=pl.BlockSpec((TM, D), lambda i: (i, 0)),
        out_shape=jax.ShapeDtypeStruct((T, D), jnp.float32),
    )(h2, x2, comb, Wg.astype(bf), Wu.astype(bf), Wd.astype(bf))

    return out.reshape(B, S, D), jnp.sum(cnt.reshape(NT, E), axis=0)


# speedup vs baseline: 2.3444x; 2.3444x over previous
import jax
import jax.numpy as jnp
from jax.experimental import pallas as pl
from jax.experimental.pallas import tpu as pltpu

B, S, D = 1, 2048, 768
H, DH = 12, 64
LAT, RD = 384, 64
E, K, DFF = 8, 2, 512
T = B * S

TM = 256
TQ = 512
NT = T // TM
NQ = T // TQ


def _dot16(a, bm):
    if a.dtype != jnp.bfloat16:
        a = a.astype(jnp.bfloat16)
    if bm.dtype != jnp.bfloat16:
        bm = bm.astype(jnp.bfloat16)
    return jnp.dot(a, bm, preferred_element_type=jnp.float32)


def _ln_apply(xv, mu, var, w, b):
    return (xv - mu) / jnp.sqrt(var + 1e-5) * w + b


def _proj_in_kernel(x_ref, mu_ref, var_ref, w_ref, b_ref, wcat_ref,
                    wkvkr_ref, wqqr_ref, okv_ref, oq_ref):
    h = _ln_apply(x_ref[...], mu_ref[...], var_ref[...], w_ref[...],
                  b_ref[...])
    c = _dot16(h, wcat_ref[...])
    okv_ref[...] = _dot16(c[:, :LAT], wkvkr_ref[...])
    oq_ref[...] = _dot16(c[:, LAT:], wqqr_ref[...])


def _flash_kernel(q_ref, k_ref, v_ref, o_ref, p_buf):
    i = pl.program_id(1)
    q = q_ref[0]
    rows = i * TQ + jax.lax.broadcasted_iota(jnp.int32, (TQ, TQ), 0)

    def pass_a(j, m):
        kb = k_ref[0, pl.ds(j * TQ, TQ), :]
        s = jax.lax.dot_general(q.astype(jnp.bfloat16),
                                kb.astype(jnp.bfloat16),
                                (((1,), (1,)), ((), ())),
                                preferred_element_type=jnp.float32)
        s = s * (DH ** -0.5)
        cols = j * TQ + jax.lax.broadcasted_iota(jnp.int32, (TQ, TQ), 1)
        s = jnp.where(cols <= rows, s, -1e30)
        p_buf[j] = s
        return jnp.maximum(m, jnp.max(s, axis=-1, keepdims=True))

    m = jax.lax.fori_loop(0, i + 1, pass_a,
                          jnp.full((TQ, 1), -1e30, jnp.float32))

    def pass_b(j, l):
        p = jnp.exp(p_buf[j] - m)
        p_buf[j] = p
        return l + jnp.sum(p, axis=-1, keepdims=True)

    l = jax.lax.fori_loop(0, i + 1, pass_b, jnp.zeros((TQ, 1), jnp.float32))

    def pass_c(j, acc):
        vb = v_ref[0, pl.ds(j * TQ, TQ), :]
        pn = p_buf[j] / l
        return acc + _dot16(pn, vb)

    acc = jax.lax.fori_loop(0, i + 1, pass_c,
                            jnp.zeros((TQ, 128), jnp.float32))
    o_ref[0] = acc[:, :DH]


def _attnout_kernel(att_ref, x_ref, wout_ref, x2_ref):
    x2_ref[...] = x_ref[...] + _dot16(att_ref[...], wout_ref[...])


def _router_kernel(x2_ref, mu_ref, var_ref, w2_ref, b2_ref, ct_ref,
                   h2_ref, comb_ref, cnt_ref):
    h2 = _ln_apply(x2_ref[...], mu_ref[...], var_ref[...], w2_ref[...],
                   b2_ref[...])
    h2_ref[...] = h2
    logits = _dot16(h2, ct_ref[...])[:, :E]
    aff = jax.nn.sigmoid(logits)
    cols = jax.lax.broadcasted_iota(jnp.int32, (TM, E), 1)
    big = jnp.int32(E)
    v1 = jnp.max(aff, axis=-1, keepdims=True)
    i1 = jnp.min(jnp.where(aff == v1, cols, big), axis=-1, keepdims=True)
    aff2 = jnp.where(cols == i1, -1.0, aff)
    v2 = jnp.max(aff2, axis=-1, keepdims=True)
    i2 = jnp.min(jnp.where(aff2 == v2, cols, big), axis=-1, keepdims=True)
    e2 = jnp.exp(v2 - v1)
    w1 = 1.0 / (1.0 + e2)
    w2 = e2 / (1.0 + e2)
    oh1 = (cols == i1)
    oh2 = (cols == i2)
    comb_ref[...] = (oh1.astype(jnp.float32) * w1 +
                     oh2.astype(jnp.float32) * w2)
    cnt_ref[0] = jnp.sum(oh1.astype(jnp.int32) + oh2.astype(jnp.int32),
                         axis=0, keepdims=True)


def _moe_kernel(h2_ref, x2_ref, comb_ref, wg_ref, wu_ref, wd_ref, o_ref):
    h2 = h2_ref[...]
    acc = x2_ref[...]
    comb = comb_ref[...]
    for e in range(E):
        g = _dot16(h2, wg_ref[e])
        u = _dot16(h2, wu_ref[e])
        a = g * jax.nn.sigmoid(g) * u
        y = _dot16(a, wd_ref[e])
        acc = acc + comb[:, e:e + 1] * y
    o_ref[...] = acc


def _rope(xv, cosv, sinv):
    xe = xv[..., 0::2]
    xo = xv[..., 1::2]
    re = xe * cosv - xo * sinv
    ro = xe * sinv + xo * cosv
    return jnp.stack([re, ro], axis=-1).reshape(xv.shape)


def _row_stats(xv):
    mu = jnp.mean(xv, axis=-1, keepdims=True)
    var = jnp.mean((xv - mu) ** 2, axis=-1, keepdims=True)
    return mu, var


@jax.jit
def kernel(x, ln1_w, ln1_b, ln2_w, ln2_b, W_ckv, W_kv, W_cq, W_q, W_qr,
           W_kr, W_out, centroids, Wg, Wu, Wd):
    xf = x.reshape(T, D)
    bf = jnp.bfloat16
    w_cat = jnp.concatenate([W_ckv, W_cq], axis=1).astype(bf)
    w_kvkr = jnp.concatenate(
        [W_kv, W_kr, jnp.zeros((LAT, 128 - RD), jnp.float32)],
        axis=1).astype(bf)
    w_qqr = jnp.concatenate([W_q, W_qr], axis=1).astype(bf)
    mu1, var1 = _row_stats(xf)

    okv, oq = pl.pallas_call(
        _proj_in_kernel,
        grid=(NT,),
        in_specs=[
            pl.BlockSpec((TM, D), lambda i: (i, 0)),
            pl.BlockSpec((TM, 1), lambda i: (i, 0)),
            pl.BlockSpec((TM, 1), lambda i: (i, 0)),
            pl.BlockSpec((1, D), lambda i: (0, 0)),
            pl.BlockSpec((1, D), lambda i: (0, 0)),
            pl.BlockSpec((D, 2 * LAT), lambda i: (0, 0)),
            pl.BlockSpec((LAT, 2 * D + 128), lambda i: (0, 0)),
            pl.BlockSpec((LAT, D + H * RD), lambda i: (0, 0)),
        ],
        out_specs=[
            pl.BlockSpec((TM, 2 * D + 128), lambda i: (i, 0)),
            pl.BlockSpec((TM, D + H * RD), lambda i: (i, 0)),
        ],
        out_shape=[
            jax.ShapeDtypeStruct((T, 2 * D + 128), jnp.float32),
            jax.ShapeDtypeStruct((T, D + H * RD), jnp.float32),
        ],
        compiler_params=pltpu.CompilerParams(
            dimension_semantics=("parallel",)),
    )(xf, mu1, var1, ln1_w.reshape(1, D), ln1_b.reshape(1, D), w_cat,
      w_kvkr, w_qqr)

    kv = okv[:, :2 * D]
    kr = okv[:, 2 * D:2 * D + RD]
    k_c = kv[:, :D].reshape(T, H, DH)
    v = kv[:, D:].reshape(T, H, DH)
    q_c = oq[:, :D].reshape(T, H, DH)
    q_r = oq[:, D:].reshape(T, H, RD)

    inv_freq = 1.0 / (10000.0 ** (jnp.arange(0, RD, 2, dtype=jnp.float32) / RD))
    freqs = jnp.outer(jnp.arange(S, dtype=jnp.float32), inv_freq)
    cosv = jnp.cos(freqs)
    sinv = jnp.sin(freqs)

    q_rot = _rope(q_r, cosv[:, None, :], sinv[:, None, :])
    k_rot = _rope(kr, cosv, sinv)

    q = jnp.concatenate([q_c, q_rot], axis=-1).transpose(1, 0, 2)
    k = jnp.concatenate(
        [k_c, jnp.broadcast_to(k_rot[:, None, :], (T, H, RD))],
        axis=-1).transpose(1, 0, 2)
    vt = jnp.concatenate(
        [v.transpose(1, 0, 2), jnp.zeros((H, T, 128 - DH), jnp.float32)],
        axis=-1)

    att = pl.pallas_call(
        _flash_kernel,
        grid=(H, NQ),
        in_specs=[
            pl.BlockSpec((1, TQ, DH + RD), lambda h, i: (h, i, 0)),
            pl.BlockSpec((1, T, DH + RD), lambda h, i: (h, 0, 0)),
            pl.BlockSpec((1, T, 128), lambda h, i: (h, 0, 0)),
        ],
        out_specs=pl.BlockSpec((1, TQ, DH), lambda h, i: (h, i, 0)),
        out_shape=jax.ShapeDtypeStruct((H, T, DH), jnp.float32),
        scratch_shapes=[pltpu.VMEM((NQ, TQ, TQ), jnp.float32)],
        compiler_params=pltpu.CompilerParams(
            dimension_semantics=("parallel", "arbitrary")),
    )(q, k, vt)

    att2d = att.transpose(1, 0, 2).reshape(T, D)

    x2 = pl.pallas_call(
        _attnout_kernel,
        grid=(NT,),
        in_specs=[
            pl.BlockSpec((TM, D), lambda i: (i, 0)),
            pl.BlockSpec((TM, D), lambda i: (i, 0)),
            pl.BlockSpec((D, D), lambda i: (0, 0)),
        ],
        out_specs=pl.BlockSpec((TM, D), lambda i: (i, 0)),
        out_shape=jax.ShapeDtypeStruct((T, D), jnp.float32),
        compiler_params=pltpu.CompilerParams(
            dimension_semantics=("parallel",)),
    )(att2d, xf, W_out.astype(bf))

    mu2, var2 = _row_stats(x2)
    ct_pad = jnp.concatenate(
        [centroids.T, jnp.zeros((D, 128 - E), jnp.float32)],
        axis=1).astype(bf)

    h2, comb, cnt = pl.pallas_call(
        _router_kernel,
        grid=(NT,),
        in_specs=[
            pl.BlockSpec((TM, D), lambda i: (i, 0)),
            pl.BlockSpec((TM, 1), lambda i: (i, 0)),
            pl.BlockSpec((TM, 1), lambda i: (i, 0)),
            pl.BlockSpec((1, D), lambda i: (0, 0)),
            pl.BlockSpec((1, D), lambda i: (0, 0)),
            pl.BlockSpec((D, 128), lambda i: (0, 0)),
        ],
        out_specs=[
            pl.BlockSpec((TM, D), lambda i: (i, 0)),
            pl.BlockSpec((TM, E), lambda i: (i, 0)),
            pl.BlockSpec((1, 1, E), lambda i: (i, 0, 0)),
        ],
        out_shape=[
            jax.ShapeDtypeStruct((T, D), jnp.float32),
            jax.ShapeDtypeStruct((T, E), jnp.float32),
            jax.ShapeDtypeStruct((NT, 1, E), jnp.int32),
        ],
        compiler_params=pltpu.CompilerParams(
            dimension_semantics=("parallel",)),
    )(x2, mu2, var2, ln2_w.reshape(1, D), ln2_b.reshape(1, D), ct_pad)

    out = pl.pallas_call(
        _moe_kernel,
        grid=(NT,),
        in_specs=[
            pl.BlockSpec((TM, D), lambda i: (i, 0)),
            pl.BlockSpec((TM, D), lambda i: (i, 0)),
            pl.BlockSpec((TM, E), lambda i: (i, 0)),
            pl.BlockSpec((E, D, DFF), lambda i: (0, 0, 0)),
            pl.BlockSpec((E, D, DFF), lambda i: (0, 0, 0)),
            pl.BlockSpec((E, DFF, D), lambda i: (0, 0, 0)),
        ],
        out_specs=pl.BlockSpec((TM, D), lambda i: (i, 0)),
        out_shape=jax.ShapeDtypeStruct((T, D), jnp.float32),
        compiler_params=pltpu.CompilerParams(
            dimension_semantics=("parallel",)),
    )(h2, x2, comb, Wg.astype(bf), Wu.astype(bf), Wd.astype(bf))

    return out.reshape(B, S, D), jnp.sum(cnt.reshape(NT, E), axis=0)
